# baseline (speedup 1.0000x reference)
import functools

import jax
import jax.numpy as jnp
from jax import lax
from jax.experimental import pallas as pl
from jax.experimental.pallas import tpu as pltpu
from jax.experimental.pallas import tpu_sc as plsc

D = 128
LANES = 16
NC = 2
NS = 16
NW = NC * NS
CHUNK = 128
NBUF = 2
NBE = 4
D_SUB = D // LANES


def _sc_spmm(edges, vals, embeds, n_chunks):
    n_real = embeds.shape[0]
    rows_per_tile = -(-n_real // (NS * 8)) * 8
    n = rows_per_tile * NS

    mesh = plsc.VectorSubcoreMesh(core_axis_name="c", subcore_axis_name="s")

    @functools.partial(
        pl.kernel,
        mesh=mesh,
        out_type=jax.ShapeDtypeStruct((NC, n, D), jnp.float32),
        scratch_types=[
            pltpu.VMEM((NBE, 2, CHUNK), jnp.int32),
            pltpu.VMEM((NBE, CHUNK), jnp.float32),
            pltpu.VMEM((NBUF, CHUNK, D), jnp.float32),
            pltpu.VMEM_SHARED((n, D), jnp.float32),
            pltpu.SemaphoreType.DMA((NBE,)),
            pltpu.SemaphoreType.DMA((NBUF,)),
            pltpu.SemaphoreType.DMA((NBUF,)),
        ],
    )
    def k(edges_hbm, vals_hbm, embeds_hbm, out_hbm,
          ibuf, vbuf, gbuf, accum, esem, gsem, ssem):
        c = lax.axis_index("c")
        s = lax.axis_index("s")
        wid = c * NS + s

        zbuf = gbuf.at[0]
        def zero_row(i, carry):
            for d in range(D_SUB):
                zbuf[i, pl.ds(d * LANES, LANES)] = jnp.zeros(
                    (LANES,), jnp.float32)
            return carry
        lax.fori_loop(0, CHUNK, zero_row, 0)

        r0 = s * rows_per_tile
        full, rem = divmod(rows_per_tile, CHUNK)
        for b in range(full):
            pltpu.sync_copy(zbuf, accum.at[pl.ds(r0 + b * CHUNK, CHUNK)])
        if rem:
            pltpu.sync_copy(zbuf.at[pl.ds(0, rem)],
                            accum.at[pl.ds(r0 + full * CHUNK, rem)])
        plsc.subcore_barrier()

        def edge_descs(t, be):
            return (
                pltpu.make_async_copy(
                    edges_hbm.at[wid, t], ibuf.at[be], esem.at[be]),
                pltpu.make_async_copy(
                    vals_hbm.at[wid, t], vbuf.at[be], esem.at[be]),
            )

        def gather_desc(t, bg):
            be = lax.rem(t, NBE)
            return pltpu.make_async_copy(
                embeds_hbm.at[ibuf.at[be, 0]], gbuf.at[bg], gsem.at[bg])

        def scatter_desc(t, bg):
            be = lax.rem(t, NBE)
            return pltpu.make_async_copy(
                gbuf.at[bg], accum.at[ibuf.at[be, 1]], ssem.at[bg])

        for t0 in range(min(3, n_chunks)):
            for d_ in edge_descs(t0, t0 % NBE):
                d_.start()
        for d_ in edge_descs(0, 0):
            d_.wait()
        gather_desc(0, 0).start()

        def chunk_body(t, carry):
            bg = lax.rem(t, NBUF)
            bg1 = lax.rem(t + 1, NBUF)

            @pl.when(t >= 1)
            def _():
                scatter_desc(t - 1, bg1).wait()

            @pl.when(t + 3 < n_chunks)
            def _():
                for d_ in edge_descs(t + 3, lax.rem(t + 3, NBE)):
                    d_.start()

            @pl.when(t + 1 < n_chunks)
            def _():
                for d_ in edge_descs(t + 1, lax.rem(t + 1, NBE)):
                    d_.wait()
                gather_desc(t + 1, bg1).start()

            gather_desc(t, bg).wait()

            buf = gbuf.at[bg]
            be = lax.rem(t, NBE)
            def scale_group(g, inner):
                base = g * LANES
                v16 = vbuf[be, pl.ds(base, LANES)]
                for l in range(LANES):
                    vb = jnp.full((LANES,), v16[l], dtype=jnp.float32)
                    e = base + l
                    for d in range(D_SUB):
                        sl = pl.ds(d * LANES, LANES)
                        buf[e, sl] = buf[e, sl] * vb
                return inner
            lax.fori_loop(0, CHUNK // LANES, scale_group, 0)

            scatter_desc(t, bg).start(add=True)
            return carry
        lax.fori_loop(0, n_chunks, chunk_body, 0)

        scatter_desc(n_chunks - 1, (n_chunks - 1) % NBUF).wait()

        plsc.subcore_barrier()
        pltpu.sync_copy(accum.at[pl.ds(r0, rows_per_tile)],
                        out_hbm.at[c, pl.ds(r0, rows_per_tile)])

    return k(edges, vals, embeds)


def _combine_body(p_ref, o_ref):
    o_ref[...] = p_ref[0] + p_ref[1]


def _combine(partials, n):
    d = partials.shape[2]
    blk = 1000
    return pl.pallas_call(
        _combine_body,
        grid=(n // blk,),
        in_specs=[pl.BlockSpec((NC, blk, d), lambda i: (0, i, 0))],
        out_specs=pl.BlockSpec((blk, d), lambda i: (i, 0)),
        out_shape=jax.ShapeDtypeStruct((n, d), jnp.float32),
    )(partials)


@jax.jit
def kernel(edge_index, values, embeds):
    n = embeds.shape[0]
    e = values.shape[0]
    rows = edge_index[0].astype(jnp.int32)
    cols = edge_index[1].astype(jnp.int32)
    vals = values.astype(jnp.float32)

    per_tile = NW * CHUNK
    n_chunks = -(-e // per_tile)
    e_pad = n_chunks * per_tile
    pad = e_pad - e
    if pad:
        pad_idx = (jnp.arange(pad, dtype=jnp.int32) * 17) % n
        rows = jnp.concatenate([rows, pad_idx])
        cols = jnp.concatenate([cols, pad_idx])
        vals = jnp.concatenate([vals, jnp.zeros((pad,), jnp.float32)])

    cols = cols.reshape(NW, n_chunks, 1, CHUNK)
    rows = rows.reshape(NW, n_chunks, 1, CHUNK)
    edges = jnp.concatenate([cols, rows], axis=2)
    vals = vals.reshape(NW, n_chunks, CHUNK)

    partials = _sc_spmm(edges, vals, embeds, n_chunks)
    return _combine(partials, n)


# speedup vs baseline: 1.0054x; 1.0054x over previous
import functools

import jax
import jax.numpy as jnp
from jax import lax
from jax.experimental import pallas as pl
from jax.experimental.pallas import tpu as pltpu
from jax.experimental.pallas import tpu_sc as plsc

D = 128
LANES = 16
NC = 2
NS = 16
NW = NC * NS
CHUNK = 128
NBUF = 2
NBE = 4
D_SUB = D // LANES


def _sc_spmm(edges, vals, embeds, n_chunks):
    n_real = embeds.shape[0]
    rows_per_tile = -(-n_real // (NS * 8)) * 8
    n = rows_per_tile * NS

    mesh = plsc.VectorSubcoreMesh(core_axis_name="c", subcore_axis_name="s")

    @functools.partial(
        pl.kernel,
        mesh=mesh,
        out_type=jax.ShapeDtypeStruct((NC, n, D), jnp.float32),
        scratch_types=[
            pltpu.VMEM((NBE, 2, CHUNK), jnp.int32),
            pltpu.VMEM((NBE, CHUNK), jnp.float32),
            pltpu.VMEM((NBUF, CHUNK, D), jnp.float32),
            pltpu.VMEM_SHARED((n, D), jnp.float32),
            pltpu.SemaphoreType.DMA((NBE,)),
            pltpu.SemaphoreType.DMA((NBUF,)),
        ],
    )
    def k(edges_hbm, vals_hbm, embeds_hbm, out_hbm,
          ibuf, vbuf, gbuf, accum, esem, gsem):
        c = lax.axis_index("c")
        s = lax.axis_index("s")
        wid = c * NS + s

        zbuf = gbuf.at[0]
        def zero_row(i, carry):
            for d in range(D_SUB):
                zbuf[i, pl.ds(d * LANES, LANES)] = jnp.zeros(
                    (LANES,), jnp.float32)
            return carry
        lax.fori_loop(0, CHUNK, zero_row, 0)

        r0 = s * rows_per_tile
        full, rem = divmod(rows_per_tile, CHUNK)
        for b in range(full):
            pltpu.sync_copy(zbuf, accum.at[pl.ds(r0 + b * CHUNK, CHUNK)])
        if rem:
            pltpu.sync_copy(zbuf.at[pl.ds(0, rem)],
                            accum.at[pl.ds(r0 + full * CHUNK, rem)])
        plsc.subcore_barrier()

        def edge_descs(t, be):
            return (
                pltpu.make_async_copy(
                    edges_hbm.at[wid, t], ibuf.at[be], esem.at[be]),
                pltpu.make_async_copy(
                    vals_hbm.at[wid, t], vbuf.at[be], esem.at[be]),
            )

        def gather_desc(t, bg):
            be = lax.rem(t, NBE)
            return pltpu.make_async_copy(
                embeds_hbm.at[ibuf.at[be, 0]], gbuf.at[bg], gsem.at[bg])

        for t0 in range(2):
            for d_ in edge_descs(t0, t0 % NBE):
                d_.start()
        for d_ in edge_descs(0, 0):
            d_.wait()
        gather_desc(0, 0).start()

        def chunk_body(t, carry):
            bg = lax.rem(t, NBUF)
            bg1 = lax.rem(t + 1, NBUF)

            for d_ in edge_descs(t + 1, lax.rem(t + 1, NBE)):
                d_.wait()
            gather_desc(t + 1, bg1).start()
            for d_ in edge_descs(t + 2, lax.rem(t + 2, NBE)):
                d_.start()

            gather_desc(t, bg).wait()

            buf = gbuf.at[bg]
            be = lax.rem(t, NBE)
            def scale_group(g, inner):
                base = g * LANES
                v16 = vbuf[be, pl.ds(base, LANES)]
                for l in range(LANES):
                    vb = jnp.full((LANES,), v16[l], dtype=jnp.float32)
                    e = base + l
                    for d in range(D_SUB):
                        sl = pl.ds(d * LANES, LANES)
                        buf[e, sl] = buf[e, sl] * vb
                return inner
            lax.fori_loop(0, CHUNK // LANES, scale_group, 0)

            pltpu.sync_copy(gbuf.at[bg], accum.at[ibuf.at[be, 1]], add=True)
            return carry
        lax.fori_loop(0, n_chunks - 1, chunk_body, 0)

        tl = n_chunks - 1
        bgl = tl % NBUF
        bel = tl % NBE
        gather_desc(tl, bgl).wait()
        bufl = gbuf.at[bgl]
        def scale_group_l(g, inner):
            base = g * LANES
            v16 = vbuf[bel, pl.ds(base, LANES)]
            for l in range(LANES):
                vb = jnp.full((LANES,), v16[l], dtype=jnp.float32)
                e = base + l
                for d in range(D_SUB):
                    sl = pl.ds(d * LANES, LANES)
                    bufl[e, sl] = bufl[e, sl] * vb
            return inner
        lax.fori_loop(0, CHUNK // LANES, scale_group_l, 0)
        pltpu.sync_copy(gbuf.at[bgl], accum.at[ibuf.at[bel, 1]], add=True)

        for d_ in edge_descs(n_chunks, n_chunks % NBE):
            d_.wait()

        plsc.subcore_barrier()
        pltpu.sync_copy(accum.at[pl.ds(r0, rows_per_tile)],
                        out_hbm.at[c, pl.ds(r0, rows_per_tile)])

    return k(edges, vals, embeds)


def _combine_body(p_ref, o_ref):
    o_ref[...] = p_ref[0] + p_ref[1]


def _combine(partials, n):
    d = partials.shape[2]
    blk = 1000
    return pl.pallas_call(
        _combine_body,
        grid=(n // blk,),
        in_specs=[pl.BlockSpec((NC, blk, d), lambda i: (0, i, 0))],
        out_specs=pl.BlockSpec((blk, d), lambda i: (i, 0)),
        out_shape=jax.ShapeDtypeStruct((n, d), jnp.float32),
    )(partials)


@jax.jit
def kernel(edge_index, values, embeds):
    n = embeds.shape[0]
    e = values.shape[0]
    rows = edge_index[0].astype(jnp.int32)
    cols = edge_index[1].astype(jnp.int32)
    vals = values.astype(jnp.float32)

    per_tile = NW * CHUNK
    n_chunks = -(-e // per_tile)
    e_pad = n_chunks * per_tile
    pad = e_pad - e
    if pad:
        pad_idx = (jnp.arange(pad, dtype=jnp.int32) * 17) % n
        rows = jnp.concatenate([rows, pad_idx])
        cols = jnp.concatenate([cols, pad_idx])
        vals = jnp.concatenate([vals, jnp.zeros((pad,), jnp.float32)])

    cols = cols.reshape(NW, n_chunks, 1, CHUNK)
    rows = rows.reshape(NW, n_chunks, 1, CHUNK)
    edges = jnp.concatenate([cols, rows], axis=2)
    vals = vals.reshape(NW, n_chunks, CHUNK)
    edges = jnp.pad(edges, ((0, 0), (0, 1), (0, 0), (0, 0)))
    vals = jnp.pad(vals, ((0, 0), (0, 1), (0, 0)))

    partials = _sc_spmm(edges, vals, embeds, n_chunks)
    return _combine(partials, n)


# speedup vs baseline: 1.7972x; 1.7875x over previous
import functools

import jax
import jax.numpy as jnp
from jax import lax
from jax.experimental import pallas as pl
from jax.experimental.pallas import tpu as pltpu
from jax.experimental.pallas import tpu_sc as plsc

D = 128
LANES = 16
NC = 2
NS = 16
NW = NC * NS
CHUNK = 128
NBUF = 2
NBE = 4
D_SUB = D // LANES


def _sc_spmm(edges, embeds, n_chunks):
    n_real = embeds.shape[0]
    rows_per_tile = -(-n_real // (NS * 8)) * 8
    n = rows_per_tile * NS

    mesh = plsc.VectorSubcoreMesh(core_axis_name="c", subcore_axis_name="s")

    @functools.partial(
        pl.kernel,
        mesh=mesh,
        out_type=jax.ShapeDtypeStruct((NC, n, D), jnp.float32),
        scratch_types=[
            pltpu.VMEM((NBE, 3, CHUNK), jnp.int32),
            pltpu.VMEM((NBUF, CHUNK, D), jnp.float32),
            pltpu.VMEM_SHARED((n, D), jnp.float32),
            pltpu.SemaphoreType.DMA((NBE,)),
            pltpu.SemaphoreType.DMA((NBUF,)),
        ],
    )
    def k(edges_hbm, embeds_hbm, out_hbm,
          ibuf, gbuf, accum, esem, gsem):
        c = lax.axis_index("c")
        s = lax.axis_index("s")
        wid = c * NS + s

        zbuf = gbuf.at[0]
        def zero_row(i, carry):
            for d in range(D_SUB):
                zbuf[i, pl.ds(d * LANES, LANES)] = jnp.zeros(
                    (LANES,), jnp.float32)
            return carry
        lax.fori_loop(0, CHUNK, zero_row, 0)

        r0 = s * rows_per_tile
        full, rem = divmod(rows_per_tile, CHUNK)
        for b in range(full):
            pltpu.sync_copy(zbuf, accum.at[pl.ds(r0 + b * CHUNK, CHUNK)])
        if rem:
            pltpu.sync_copy(zbuf.at[pl.ds(0, rem)],
                            accum.at[pl.ds(r0 + full * CHUNK, rem)])
        plsc.subcore_barrier()

        def edge_descs(t, be):
            return (
                pltpu.make_async_copy(
                    edges_hbm.at[wid, t], ibuf.at[be], esem.at[be]),
            )

        def gather_desc(t, bg):
            be = lax.rem(t, NBE)
            return pltpu.make_async_copy(
                embeds_hbm.at[ibuf.at[be, 0]], gbuf.at[bg], gsem.at[bg])

        for t0 in range(2):
            for d_ in edge_descs(t0, t0 % NBE):
                d_.start()

        def chunk_body(t, carry):
            for d_ in edge_descs(t + 2, lax.rem(t + 2, NBE)):
                d_.start()
            be = lax.rem(t, NBE)
            for d_ in edge_descs(t, be):
                d_.wait()

            gather_desc(t, 0).start()
            gather_desc(t, 0).wait()

            buf = gbuf.at[0]
            def scale_group(g, inner):
                base = g * LANES
                v16 = lax.bitcast_convert_type(
                    ibuf[be, 2, pl.ds(base, LANES)], jnp.float32)
                for l in range(LANES):
                    vb = jnp.full((LANES,), v16[l], dtype=jnp.float32)
                    e = base + l
                    for d in range(D_SUB):
                        sl = pl.ds(d * LANES, LANES)
                        buf[e, sl] = buf[e, sl] * vb
                return inner
            lax.fori_loop(0, CHUNK // LANES, scale_group, 0)

            pltpu.sync_copy(gbuf.at[0], accum.at[ibuf.at[be, 1]], add=True)
            return carry
        lax.fori_loop(0, n_chunks, chunk_body, 0)

        for td in (n_chunks, n_chunks + 1):
            for d_ in edge_descs(td, td % NBE):
                d_.wait()

        plsc.subcore_barrier()
        pltpu.sync_copy(accum.at[pl.ds(r0, rows_per_tile)],
                        out_hbm.at[c, pl.ds(r0, rows_per_tile)])

    return k(edges, embeds)


def _combine_body(p_ref, o_ref):
    o_ref[...] = p_ref[0] + p_ref[1]


def _combine(partials, n):
    d = partials.shape[2]
    blk = 1000
    return pl.pallas_call(
        _combine_body,
        grid=(n // blk,),
        in_specs=[pl.BlockSpec((NC, blk, d), lambda i: (0, i, 0))],
        out_specs=pl.BlockSpec((blk, d), lambda i: (i, 0)),
        out_shape=jax.ShapeDtypeStruct((n, d), jnp.float32),
    )(partials)


@jax.jit
def kernel(edge_index, values, embeds):
    n = embeds.shape[0]
    e = values.shape[0]
    rows = edge_index[0].astype(jnp.int32)
    cols = edge_index[1].astype(jnp.int32)
    vals = values.astype(jnp.float32)

    per_tile = NW * CHUNK
    n_chunks = -(-e // per_tile)
    e_pad = n_chunks * per_tile
    pad = e_pad - e
    if pad:
        pad_idx = (jnp.arange(pad, dtype=jnp.int32) * 17) % n
        rows = jnp.concatenate([rows, pad_idx])
        cols = jnp.concatenate([cols, pad_idx])
        vals = jnp.concatenate([vals, jnp.zeros((pad,), jnp.float32)])

    cols = cols.reshape(NW, n_chunks, 1, CHUNK)
    rows = rows.reshape(NW, n_chunks, 1, CHUNK)
    vals_i = lax.bitcast_convert_type(vals, jnp.int32)
    vals_i = vals_i.reshape(NW, n_chunks, 1, CHUNK)
    edges = jnp.concatenate([cols, rows, vals_i], axis=2)
    edges = jnp.pad(edges, ((0, 0), (0, 2), (0, 0), (0, 0)))

    partials = _sc_spmm(edges, embeds, n_chunks)
    return _combine(partials, n)
